# baseline (speedup 1.0000x reference)
import jax
import jax.numpy as jnp
from jax.experimental import pallas as pl
from jax.experimental.pallas import tpu as pltpu

DESIRED = 5.0
UNDESIRED = 1.0


def _body(x_ref, t_ref, o_ref, *, scale):
    x = x_ref[...]
    t = t_ref[...]
    oc = jnp.sum(x, axis=0, keepdims=True)
    oc = jnp.where((t == DESIRED) & (oc > DESIRED), DESIRED, oc)
    oc = jnp.where((t == UNDESIRED) & (oc < UNDESIRED), UNDESIRED, oc)
    d = oc - t
    part = jnp.sum(d * d) * scale

    @pl.when(pl.program_id(0) == 0)
    def _():
        o_ref[0, 0] = 0.0

    o_ref[0, 0] += part


def kernel(output, target):
    T, N, C = output.shape
    NC = N * C
    x2 = output.reshape(T, NC)
    t2 = target.reshape(1, NC)
    CH = 6400
    assert NC % CH == 0
    K = NC // CH
    scale = 0.5 / T

    import functools
    out = pl.pallas_call(
        functools.partial(_body, scale=scale),
        grid=(K,),
        in_specs=[
            pl.BlockSpec((T, CH), lambda i: (0, i)),
            pl.BlockSpec((1, CH), lambda i: (0, i)),
        ],
        out_specs=pl.BlockSpec((1, 1), lambda i: (0, 0),
                               memory_space=pltpu.SMEM),
        out_shape=jax.ShapeDtypeStruct((1, 1), jnp.float32),
    )(x2, t2)
    return out[0, 0]


# speedup vs baseline: 2.3193x; 2.3193x over previous
import functools

import jax
import jax.numpy as jnp
from jax.experimental import pallas as pl
from jax.experimental.pallas import tpu as pltpu

DESIRED = 5.0
UNDESIRED = 1.0


def _body(x_ref, t_ref, o_ref, acc_ref, *, nsteps, scale):
    j = pl.program_id(0)
    s = jnp.sum(x_ref[...], axis=0)

    @pl.when(j == 0)
    def _():
        acc_ref[...] = s

    @pl.when(j > 0)
    def _():
        acc_ref[...] += s

    @pl.when(j == nsteps - 1)
    def _():
        t = t_ref[...]
        oc = acc_ref[...]
        oc = jnp.where((t == DESIRED) & (oc > DESIRED), DESIRED, oc)
        oc = jnp.where((t == UNDESIRED) & (oc < UNDESIRED), UNDESIRED, oc)
        d = oc - t
        o_ref[0, 0] = jnp.sum(d * d) * scale


def kernel(output, target):
    T, N, C = output.shape
    TB = 4
    assert T % TB == 0
    K = T // TB
    scale = 0.5 / T

    out = pl.pallas_call(
        functools.partial(_body, nsteps=K, scale=scale),
        grid=(K,),
        in_specs=[
            pl.BlockSpec((TB, N, C), lambda j: (j, 0, 0)),
            pl.BlockSpec((N, C), lambda j: (0, 0)),
        ],
        out_specs=pl.BlockSpec((1, 1), lambda j: (0, 0),
                               memory_space=pltpu.SMEM),
        out_shape=jax.ShapeDtypeStruct((1, 1), jnp.float32),
        scratch_shapes=[pltpu.VMEM((N, C), jnp.float32)],
    )(output, target)
    return out[0, 0]
